# baseline (speedup 1.0000x reference)
import jax
import jax.numpy as jnp
from jax.experimental import pallas as pl
from jax.experimental.pallas import tpu as pltpu

B, C, H, W, O = 32, 64, 64, 64, 128
KH = KW = 3
HP, WP = H - KH + 1, W - KW + 1
COLS = HP * W
HW = H * W
XF = HW + 128
CK = C * KH * KW
CKE = CK + 8
NCORES = 2
BPC = B // NCORES


def _hebb_kernel(x_ref, wq_ref, y_ref, a_ref, s_ref, xf_scr, pt_scr):
    j = pl.program_id(1)

    @pl.when(j == 0)
    def _tails():
        xf_scr[:, HW:XF] = jnp.zeros((C, XF - HW), jnp.float32)
        rid = jax.lax.broadcasted_iota(jnp.int32, (8, COLS), 0)
        pt_scr[CK:CKE, :] = jnp.where(rid == 0, 1.0, 0.0)

    for h in range(H):
        xf_scr[:, h * W:(h + 1) * W] = x_ref[:, h, :]

    for t in range(KH * KW):
        kh, kw = divmod(t, KW)
        off = kh * W + kw
        pt_scr[t * C:(t + 1) * C, :] = xf_scr[:, off:off + COLS]

    wq = wq_ref[...]
    lid = jax.lax.broadcasted_iota(jnp.int32, (1, CKE), 1)
    wsq = jnp.where(lid < CK, wq * wq, 0.0)
    nrm = jnp.sqrt(jnp.sum(wsq, axis=1, keepdims=True))
    rn = jnp.where(nrm == 0.0, 1.0, 1.0 / nrm)
    wn = jnp.where(lid < CK, wq * rn, wq)

    y = jnp.dot(wn, pt_scr[...], preferred_element_type=jnp.float32)

    m = jnp.max(y, axis=0, keepdims=True)
    e = jnp.exp(y - m)
    r = e / jnp.sum(e, axis=0, keepdims=True)

    cid = jax.lax.broadcasted_iota(jnp.int32, (1, COLS), 1)
    rm = jnp.where(cid % W < WP, r, 0.0)
    r2 = rm * rm

    s1 = jnp.sum(rm, axis=1, keepdims=True)
    s2 = jnp.sum(r2, axis=1, keepdims=True)
    a = jax.lax.dot_general(r2, pt_scr[...], (((1,), (1,)), ((), ())),
                            preferred_element_type=jnp.float32)

    for i in range(HP):
        y_ref[:, i, :] = y[:, i * W:i * W + WP]

    sj = jnp.concatenate([s1, s2], axis=1)

    @pl.when(j == 0)
    def _init():
        a_ref[...] = a
        s_ref[...] = sj

    @pl.when(j > 0)
    def _acc():
        a_ref[...] += a
        s_ref[...] += sj


@jax.jit
def kernel(x, weight, bias):
    wflat = weight.transpose(0, 2, 3, 1).reshape(O, CK)
    wq = jnp.concatenate(
        [wflat, bias.reshape(O, 1), jnp.zeros((O, 7), jnp.float32)], axis=1)

    y, a_part, s_part = pl.pallas_call(
        _hebb_kernel,
        grid=(NCORES, BPC),
        in_specs=[
            pl.BlockSpec((None, C, H, W), lambda i, j: (i * BPC + j, 0, 0, 0)),
            pl.BlockSpec((O, CKE), lambda i, j: (0, 0)),
        ],
        out_specs=[
            pl.BlockSpec((None, O, HP, WP), lambda i, j: (i * BPC + j, 0, 0, 0)),
            pl.BlockSpec((None, O, CKE), lambda i, j: (i, 0, 0)),
            pl.BlockSpec((None, O, 2), lambda i, j: (i, 0, 0)),
        ],
        out_shape=[
            jax.ShapeDtypeStruct((B, O, HP, WP), jnp.float32),
            jax.ShapeDtypeStruct((NCORES, O, CKE), jnp.float32),
            jax.ShapeDtypeStruct((NCORES, O, 2), jnp.float32),
        ],
        scratch_shapes=[
            pltpu.VMEM((C, XF), jnp.float32),
            pltpu.VMEM((CKE, COLS), jnp.float32),
        ],
        compiler_params=pltpu.CompilerParams(
            dimension_semantics=("parallel", "arbitrary"),
            vmem_limit_bytes=56 * 1024 * 1024,
        ),
    )(x, wq)

    a = a_part.sum(axis=0)[:, :CK]
    s1 = s_part[:, :, 0].sum(axis=0)
    s2 = s_part[:, :, 1].sum(axis=0)
    r_sum = jnp.where(s1 == 0.0, 1.0, s1)
    a = a.reshape(O, KH, KW, C).transpose(0, 3, 1, 2)
    scale = (1.0 / r_sum)[:, None, None, None]
    delta_w = a * scale - (s2[:, None, None, None] * scale) * weight
    return y, delta_w


# speedup vs baseline: 1.4352x; 1.4352x over previous
import jax
import jax.numpy as jnp
from jax.experimental import pallas as pl
from jax.experimental.pallas import tpu as pltpu

B, C, H, W, O = 32, 64, 64, 64, 128
KH = KW = 3
HP, WP = H - KH + 1, W - KW + 1
COLS = HP * W
HW = H * W
CK = C * KH * KW
CKE = CK + 8
NCORES = 2
BPC = B // NCORES


def _hebb_kernel(x_ref, wq_ref, y_ref, a_ref, s_ref, pt_scr):
    j = pl.program_id(1)

    @pl.when(j == 0)
    def _tail():
        rid = jax.lax.broadcasted_iota(jnp.int32, (8, COLS), 0)
        pt_scr[CK:CKE, :] = jnp.where(rid == 0, 1.0, 0.0)

    for t in range(KH * KW):
        kh, kw = divmod(t, KW)
        off = kh * W + kw
        n = min(HW - off, COLS)
        pt_scr[t * C:(t + 1) * C, 0:n] = x_ref[:, off:off + n]
        if n < COLS:
            pt_scr[t * C:(t + 1) * C, n:COLS] = jnp.zeros((C, COLS - n),
                                                          jnp.float32)

    wq = wq_ref[...]
    lid = jax.lax.broadcasted_iota(jnp.int32, (1, CKE), 1)
    wsq = jnp.where(lid < CK, wq * wq, 0.0)
    nrm = jnp.sqrt(jnp.sum(wsq, axis=1, keepdims=True))
    rn = jnp.where(nrm == 0.0, 1.0, 1.0 / nrm)
    wn = jnp.where(lid < CK, wq * rn, wq)

    y = jnp.dot(wn, pt_scr[...], preferred_element_type=jnp.float32)
    y_ref[...] = y

    m = jnp.max(y, axis=0, keepdims=True)
    e = jnp.exp(y - m)
    r = e / jnp.sum(e, axis=0, keepdims=True)

    cid = jax.lax.broadcasted_iota(jnp.int32, (1, COLS), 1)
    rm = jnp.where(cid % W < WP, r, 0.0)
    r2 = rm * rm

    s1 = jnp.sum(rm, axis=1, keepdims=True)
    s2 = jnp.sum(r2, axis=1, keepdims=True)
    a = jax.lax.dot_general(r2, pt_scr[...], (((1,), (1,)), ((), ())),
                            preferred_element_type=jnp.float32)

    sj = jnp.concatenate([s1, s2], axis=1)

    @pl.when(j == 0)
    def _init():
        a_ref[...] = a
        s_ref[...] = sj

    @pl.when(j > 0)
    def _acc():
        a_ref[...] += a
        s_ref[...] += sj


@jax.jit
def kernel(x, weight, bias):
    xv = x.reshape(B, C, HW)
    wflat = weight.transpose(0, 2, 3, 1).reshape(O, CK)
    wq = jnp.concatenate(
        [wflat, bias.reshape(O, 1), jnp.zeros((O, 7), jnp.float32)], axis=1)

    y_flat, a_part, s_part = pl.pallas_call(
        _hebb_kernel,
        grid=(NCORES, BPC),
        in_specs=[
            pl.BlockSpec((None, C, HW), lambda i, j: (i * BPC + j, 0, 0)),
            pl.BlockSpec((O, CKE), lambda i, j: (0, 0)),
        ],
        out_specs=[
            pl.BlockSpec((None, O, COLS), lambda i, j: (i * BPC + j, 0, 0)),
            pl.BlockSpec((None, O, CKE), lambda i, j: (i, 0, 0)),
            pl.BlockSpec((None, O, 2), lambda i, j: (i, 0, 0)),
        ],
        out_shape=[
            jax.ShapeDtypeStruct((B, O, COLS), jnp.float32),
            jax.ShapeDtypeStruct((NCORES, O, CKE), jnp.float32),
            jax.ShapeDtypeStruct((NCORES, O, 2), jnp.float32),
        ],
        scratch_shapes=[pltpu.VMEM((CKE, COLS), jnp.float32)],
        compiler_params=pltpu.CompilerParams(
            dimension_semantics=("parallel", "arbitrary"),
            vmem_limit_bytes=56 * 1024 * 1024,
        ),
    )(xv, wq)

    y = y_flat.reshape(B, O, HP, W)[:, :, :, :WP]

    a = a_part.sum(axis=0)[:, :CK]
    s1 = s_part[:, :, 0].sum(axis=0)
    s2 = s_part[:, :, 1].sum(axis=0)
    r_sum = jnp.where(s1 == 0.0, 1.0, s1)
    a = a.reshape(O, KH, KW, C).transpose(0, 3, 1, 2)
    scale = (1.0 / r_sum)[:, None, None, None]
    delta_w = a * scale - (s2[:, None, None, None] * scale) * weight
    return y, delta_w


# speedup vs baseline: 1.9115x; 1.3319x over previous
import jax
import jax.numpy as jnp
from jax.experimental import pallas as pl
from jax.experimental.pallas import tpu as pltpu

B, C, H, W, O = 32, 64, 64, 64, 128
KH = KW = 3
HP, WP = H - KH + 1, W - KW + 1
COLS = HP * W
HW = H * W
CK = C * KH * KW
CKE = CK + 8
NCORES = 2
BPC = B // NCORES


def _hebb_kernel(x_ref, wq_ref, y_ref, a_ref, s_ref, pt_scr):
    j = pl.program_id(1)

    @pl.when(j == 0)
    def _tail():
        rid = jax.lax.broadcasted_iota(jnp.int32, (8, COLS), 0)
        pt_scr[CK:CKE, :] = jnp.where(rid == 0, 1.0, 0.0).astype(jnp.bfloat16)

    for t in range(KH * KW):
        kh, kw = divmod(t, KW)
        off = kh * W + kw
        n = min(HW - off, COLS)
        pt_scr[t * C:(t + 1) * C, 0:n] = x_ref[:, off:off + n].astype(
            jnp.bfloat16)
        if n < COLS:
            pt_scr[t * C:(t + 1) * C, n:COLS] = jnp.zeros((C, COLS - n),
                                                          jnp.bfloat16)

    wq = wq_ref[...]
    lid = jax.lax.broadcasted_iota(jnp.int32, (1, CKE), 1)
    wsq = jnp.where(lid < CK, wq * wq, 0.0)
    nrm = jnp.sqrt(jnp.sum(wsq, axis=1, keepdims=True))
    rn = jnp.where(nrm == 0.0, 1.0, 1.0 / nrm)
    wn = jnp.where(lid < CK, wq * rn, wq).astype(jnp.bfloat16)

    ptb = pt_scr[...]

    y = jnp.dot(wn, ptb, preferred_element_type=jnp.float32)

    m = jnp.max(y, axis=0, keepdims=True)
    e = jnp.exp(y - m)
    r = e / jnp.sum(e, axis=0, keepdims=True)

    cid = jax.lax.broadcasted_iota(jnp.int32, (1, COLS), 1)
    rm = jnp.where(cid % W < WP, r, 0.0)
    r2 = rm * rm

    s1 = jnp.sum(rm, axis=1, keepdims=True)
    s2 = jnp.sum(r2, axis=1, keepdims=True)
    a = jax.lax.dot_general(r2.astype(jnp.bfloat16), ptb,
                            (((1,), (1,)), ((), ())),
                            preferred_element_type=jnp.float32)

    for i in range(HP):
        y_ref[:, i * WP:(i + 1) * WP] = y[:, i * W:i * W + WP]

    sj = jnp.concatenate([s1, s2], axis=1)

    @pl.when(j == 0)
    def _init():
        a_ref[...] = a
        s_ref[...] = sj

    @pl.when(j > 0)
    def _acc():
        a_ref[...] += a
        s_ref[...] += sj


@jax.jit
def kernel(x, weight, bias):
    xv = x.reshape(B, C, HW)
    wflat = weight.transpose(0, 2, 3, 1).reshape(O, CK)
    wq = jnp.concatenate(
        [wflat, bias.reshape(O, 1), jnp.zeros((O, 7), jnp.float32)], axis=1)

    y_flat, a_part, s_part = pl.pallas_call(
        _hebb_kernel,
        grid=(NCORES, BPC),
        in_specs=[
            pl.BlockSpec((None, C, HW), lambda i, j: (i * BPC + j, 0, 0)),
            pl.BlockSpec((O, CKE), lambda i, j: (0, 0)),
        ],
        out_specs=[
            pl.BlockSpec((None, O, HP * WP), lambda i, j: (i * BPC + j, 0, 0)),
            pl.BlockSpec((None, O, CKE), lambda i, j: (i, 0, 0)),
            pl.BlockSpec((None, O, 2), lambda i, j: (i, 0, 0)),
        ],
        out_shape=[
            jax.ShapeDtypeStruct((B, O, HP * WP), jnp.float32),
            jax.ShapeDtypeStruct((NCORES, O, CKE), jnp.float32),
            jax.ShapeDtypeStruct((NCORES, O, 2), jnp.float32),
        ],
        scratch_shapes=[pltpu.VMEM((CKE, COLS), jnp.bfloat16)],
        compiler_params=pltpu.CompilerParams(
            dimension_semantics=("parallel", "arbitrary"),
            vmem_limit_bytes=56 * 1024 * 1024,
        ),
    )(xv, wq)

    y = y_flat.reshape(B, O, HP, WP)

    a = a_part.sum(axis=0)[:, :CK]
    s1 = s_part[:, :, 0].sum(axis=0)
    s2 = s_part[:, :, 1].sum(axis=0)
    r_sum = jnp.where(s1 == 0.0, 1.0, s1)
    a = a.reshape(O, KH, KW, C).transpose(0, 3, 1, 2)
    scale = (1.0 / r_sum)[:, None, None, None]
    delta_w = a * scale - (s2[:, None, None, None] * scale) * weight
    return y, delta_w
